# baseline (speedup 1.0000x reference)
import jax
import jax.numpy as jnp
from jax import lax
from jax.experimental import pallas as pl


_NP = 8192
_CH = 64


def _tc_kernel(x_ref, out_ref):
    c = x_ref.shape[-1]
    iota = lax.broadcasted_iota(jnp.int32, (_CH, c), 1)

    def body(k, _):
        x = x_ref[pl.ds(k * _CH, _CH), :]
        m = jnp.max(x, axis=-1, keepdims=True)
        idx = jnp.min(jnp.where(x == m, iota, c), axis=-1, keepdims=True)
        d = iota - idx
        low = jnp.sum(jnp.where(d == -1, x, 0.0), axis=-1, keepdims=True)
        up = jnp.sum(jnp.where(d == 1, x, 0.0), axis=-1, keepdims=True)
        g = jnp.maximum(low, up)
        nbr = jnp.where(up > low, idx + 1, idx - 1).astype(jnp.float32)
        idx_f = idx.astype(jnp.float32)
        fl = (m * idx_f + g * nbr) / (m + g)
        out_ref[pl.ds(k * _CH, _CH), :] = fl * jnp.float32(0.035) - jnp.float32(4.0)
        return 0

    lax.fori_loop(0, _NP // _CH, body, 0)


def kernel(prob):
    hei, wid, cls = prob.shape
    npix = hei * wid
    x = prob.reshape(npix, cls)
    out = pl.pallas_call(
        _tc_kernel,
        grid=(npix // _NP,),
        in_specs=[pl.BlockSpec((_NP, cls), lambda i: (i, 0))],
        out_specs=pl.BlockSpec((_NP, 1), lambda i: (i, 0)),
        out_shape=jax.ShapeDtypeStruct((npix, 1), jnp.float32),
    )(x)
    return out.reshape(hei, wid)


# speedup vs baseline: 7.7015x; 7.7015x over previous
import jax
import jax.numpy as jnp
from jax import lax
from jax.experimental import pallas as pl


_BH = 16


def _disp_block(x):
    c = x.shape[-1]
    m = jnp.max(x, axis=-1)
    iota = lax.broadcasted_iota(jnp.int32, x.shape, 2).astype(jnp.float32)
    idx = jnp.min(jnp.where(x == m[..., None], iota, float(c)), axis=-1)
    idx_e = idx[..., None]
    low = jnp.sum(jnp.where(iota == idx_e - 1.0, x, 0.0), axis=-1)
    up = jnp.sum(jnp.where(iota == idx_e + 1.0, x, 0.0), axis=-1)
    g = jnp.maximum(low, up)
    nbr = jnp.where(up > low, idx + 1.0, idx - 1.0)
    fl = (m * idx + g * nbr) / (m + g)
    return fl * jnp.float32(0.035) - jnp.float32(4.0)


def _tc_kernel(prob_ref, out_ref):
    out_ref[...] = _disp_block(prob_ref[...])


def kernel(prob):
    hei, wid, cls = prob.shape
    grid = hei // _BH
    return pl.pallas_call(
        _tc_kernel,
        grid=(grid,),
        in_specs=[pl.BlockSpec((_BH, wid, cls), lambda i: (i, 0, 0))],
        out_specs=pl.BlockSpec((_BH, wid), lambda i: (i, 0)),
        out_shape=jax.ShapeDtypeStruct((hei, wid), jnp.float32),
    )(prob)
